# baseline (speedup 1.0000x reference)
import functools

import jax
import jax.numpy as jnp
from jax import lax
from jax.experimental import pallas as pl
from jax.experimental.pallas import tpu as pltpu
from jax.experimental.pallas import tpu_sc as plsc

_FIELD_DIM = 100000
_IDX_ROW = 128

_NC = 2
_NS = 16
_NW = _NC * _NS


def _sc_gather(x, table):
    batch, nf = x.shape
    emb = table.shape[1]
    rows_w = batch // _NW
    n_g = rows_w // _IDX_ROW

    mesh = plsc.VectorSubcoreMesh(core_axis_name="c", subcore_axis_name="s")

    @functools.partial(
        pl.kernel,
        mesh=mesh,
        out_type=jax.ShapeDtypeStruct((batch, nf, emb), jnp.float32),
        scratch_types=[
            pltpu.VMEM((rows_w, nf), jnp.int32),
            pltpu.VMEM((rows_w,), jnp.int32),
            pltpu.VMEM((rows_w, 1, emb), jnp.float32),
            pltpu.SemaphoreType.DMA,
        ],
        compiler_params=pltpu.CompilerParams(
            use_tc_tiling_on_sc=False, needs_layout_passes=False),
    )
    def body(x_hbm, table_hbm, out_hbm, xs_v, idx_v, rows_v, sem):
        wid = lax.axis_index("s") * _NC + lax.axis_index("c")
        b0 = wid * rows_w
        pltpu.sync_copy(x_hbm.at[pl.ds(b0, rows_w)], xs_v)

        def field_body(j, carry):
            jv = jnp.full((16,), 0, jnp.int32) + j
            for k in range(rows_w // 16):
                riv = jax.lax.iota(jnp.int32, 16) + (k * 16)
                idx_v[pl.ds(k * 16, 16)] = plsc.load_gather(xs_v, [riv, jv])
            slab = table_hbm.at[pl.ds(j * _FIELD_DIM, _FIELD_DIM)]
            copies = [
                pltpu.async_copy(
                    slab.at[idx_v.at[pl.ds(k * _IDX_ROW, _IDX_ROW)]],
                    rows_v.at[pl.ds(k * _IDX_ROW, _IDX_ROW), 0],
                    sem,
                )
                for k in range(n_g)
            ]
            for cp in copies:
                cp.wait()
            pltpu.sync_copy(rows_v, out_hbm.at[pl.ds(b0, rows_w), pl.ds(j, 1)])
            return carry

        lax.fori_loop(0, nf, field_body, 0)

    return body(x, table)


def kernel(x, table):
    return _sc_gather(x, table)
